# baseline (speedup 1.0000x reference)
import functools

import jax
import jax.numpy as jnp
from jax.experimental import pallas as pl
from jax.experimental.pallas import tpu as pltpu

_TEMPERATURE = 10.0
_BM = 4096


def _flash_body(addr_ref, keys_ref, values_ref, out_ref, m_ref, acc_ref,
                *, num_blocks):
    i = pl.program_id(0)

    @pl.when(i == 0)
    def _init():
        m_ref[...] = jnp.full_like(m_ref, -jnp.inf)
        acc_ref[...] = jnp.zeros_like(acc_ref)

    s = jax.lax.dot_general(
        addr_ref[...], keys_ref[...],
        (((1,), (1,)), ((), ())),
        preferred_element_type=jnp.float32,
    )
    m_prev = m_ref[...]
    m_new = jnp.maximum(m_prev, jnp.max(s, axis=1, keepdims=True))
    alpha = jnp.exp(m_prev - m_new)
    p = jnp.exp(s - m_new).astype(jnp.bfloat16)
    pv = jax.lax.dot_general(
        p, values_ref[...],
        (((1,), (0,)), ((), ())),
        preferred_element_type=jnp.float32,
    )
    m_ref[...] = m_new
    acc_ref[...] = acc_ref[...] * alpha + pv

    @pl.when(i == num_blocks - 1)
    def _fini():
        acc = acc_ref[...]
        out_ref[...] = acc[:, :-1] / acc[:, -1:]


@jax.jit
def kernel(keys, values, addr):
    M, D = keys.shape
    Q = addr.shape[0]
    V = values.shape[1]
    num_blocks = M // _BM

    addr_s = (addr * _TEMPERATURE).astype(jnp.bfloat16)
    keys_b = keys.astype(jnp.bfloat16)
    values_b = jnp.concatenate(
        [values, jnp.ones((M, 1), values.dtype)], axis=1
    ).astype(jnp.bfloat16)

    return pl.pallas_call(
        functools.partial(_flash_body, num_blocks=num_blocks),
        grid=(num_blocks,),
        in_specs=[
            pl.BlockSpec((Q, D), lambda i: (0, 0)),
            pl.BlockSpec((_BM, D), lambda i: (i, 0)),
            pl.BlockSpec((_BM, V + 1), lambda i: (i, 0)),
        ],
        out_specs=pl.BlockSpec((Q, V), lambda i: (0, 0)),
        out_shape=jax.ShapeDtypeStruct((Q, V), jnp.float32),
        scratch_shapes=[
            pltpu.VMEM((Q, 1), jnp.float32),
            pltpu.VMEM((Q, V + 1), jnp.float32),
        ],
        compiler_params=pltpu.CompilerParams(
            dimension_semantics=("arbitrary",),
        ),
    )(addr_s, keys_b, values_b)


# speedup vs baseline: 1.0273x; 1.0273x over previous
import functools

import jax
import jax.numpy as jnp
from jax.experimental import pallas as pl
from jax.experimental.pallas import tpu as pltpu

_TEMPERATURE = 10.0
_BM = 4096


def _flash_body(addr_ref, keys_ref, values_ref, out_ref, m_ref, acc_ref,
                *, num_blocks):
    i = pl.program_id(0)

    @pl.when(i == 0)
    def _init():
        m_ref[...] = jnp.full_like(m_ref, -jnp.inf)
        acc_ref[...] = jnp.zeros_like(acc_ref)

    addr = (addr_ref[...] * _TEMPERATURE).astype(jnp.bfloat16)
    keys = keys_ref[...].astype(jnp.bfloat16)
    vals = values_ref[...].astype(jnp.bfloat16)
    ones = jnp.ones((vals.shape[0], 1), jnp.bfloat16)
    vals1 = jnp.concatenate([vals, ones], axis=1)

    s = jax.lax.dot_general(
        addr, keys,
        (((1,), (1,)), ((), ())),
        preferred_element_type=jnp.float32,
    )
    m_prev = m_ref[...]
    m_new = jnp.maximum(m_prev, jnp.max(s, axis=1, keepdims=True))
    alpha = jnp.exp(m_prev - m_new)
    p = jnp.exp(s - m_new).astype(jnp.bfloat16)
    pv = jax.lax.dot_general(
        p, vals1,
        (((1,), (0,)), ((), ())),
        preferred_element_type=jnp.float32,
    )
    m_ref[...] = m_new
    acc_ref[...] = acc_ref[...] * alpha + pv

    @pl.when(i == num_blocks - 1)
    def _fini():
        acc = acc_ref[...]
        out_ref[...] = acc[:, :-1] / acc[:, -1:]


@jax.jit
def kernel(keys, values, addr):
    M, D = keys.shape
    Q = addr.shape[0]
    V = values.shape[1]
    num_blocks = M // _BM

    return pl.pallas_call(
        functools.partial(_flash_body, num_blocks=num_blocks),
        grid=(num_blocks,),
        in_specs=[
            pl.BlockSpec((Q, D), lambda i: (0, 0)),
            pl.BlockSpec((_BM, D), lambda i: (i, 0)),
            pl.BlockSpec((_BM, V), lambda i: (i, 0)),
        ],
        out_specs=pl.BlockSpec((Q, V), lambda i: (0, 0)),
        out_shape=jax.ShapeDtypeStruct((Q, V), jnp.float32),
        scratch_shapes=[
            pltpu.VMEM((Q, 1), jnp.float32),
            pltpu.VMEM((Q, V + 1), jnp.float32),
        ],
        compiler_params=pltpu.CompilerParams(
            dimension_semantics=("arbitrary",),
        ),
    )(addr, keys, values)


# speedup vs baseline: 1.1324x; 1.1024x over previous
import functools

import jax
import jax.numpy as jnp
from jax.experimental import pallas as pl
from jax.experimental.pallas import tpu as pltpu

_TEMPERATURE = 10.0
_BM = 4096


def _flash_body(addr_ref, keys_ref, values_ref, out_ref, m_ref, acc_ref,
                *, num_blocks):
    i = pl.program_id(0)

    @pl.when(i == 0)
    def _init():
        m_ref[...] = jnp.full_like(m_ref, -jnp.inf)
        acc_ref[...] = jnp.zeros_like(acc_ref)

    addr = (addr_ref[...] * _TEMPERATURE).astype(jnp.bfloat16)
    keys = keys_ref[...].astype(jnp.bfloat16)
    vals = values_ref[...].astype(jnp.bfloat16)
    ones = jnp.ones((vals.shape[0], 1), jnp.bfloat16)
    vals1 = jnp.concatenate([vals, ones], axis=1)

    s = jax.lax.dot_general(
        addr, keys,
        (((1,), (1,)), ((), ())),
        preferred_element_type=jnp.float32,
    ).astype(jnp.bfloat16)
    m_prev = m_ref[...]
    m_new = jnp.maximum(m_prev, jnp.max(s, axis=1, keepdims=True))
    alpha = jnp.exp(m_prev - m_new)
    p = jnp.exp(s - m_new)
    pv = jax.lax.dot_general(
        p, vals1,
        (((1,), (0,)), ((), ())),
        preferred_element_type=jnp.float32,
    )
    m_ref[...] = m_new
    acc_ref[...] = acc_ref[...] * alpha.astype(jnp.float32) + pv

    @pl.when(i == num_blocks - 1)
    def _fini():
        acc = acc_ref[...]
        out_ref[...] = acc[:, :-1] / acc[:, -1:]


@jax.jit
def kernel(keys, values, addr):
    M, D = keys.shape
    Q = addr.shape[0]
    V = values.shape[1]
    num_blocks = M // _BM

    return pl.pallas_call(
        functools.partial(_flash_body, num_blocks=num_blocks),
        grid=(num_blocks,),
        in_specs=[
            pl.BlockSpec((Q, D), lambda i: (0, 0)),
            pl.BlockSpec((_BM, D), lambda i: (i, 0)),
            pl.BlockSpec((_BM, V), lambda i: (i, 0)),
        ],
        out_specs=pl.BlockSpec((Q, V), lambda i: (0, 0)),
        out_shape=jax.ShapeDtypeStruct((Q, V), jnp.float32),
        scratch_shapes=[
            pltpu.VMEM((Q, 1), jnp.bfloat16),
            pltpu.VMEM((Q, V + 1), jnp.float32),
        ],
        compiler_params=pltpu.CompilerParams(
            dimension_semantics=("arbitrary",),
        ),
    )(addr, keys, values)


# speedup vs baseline: 1.1587x; 1.0232x over previous
import functools

import jax
import jax.numpy as jnp
from jax.experimental import pallas as pl
from jax.experimental.pallas import tpu as pltpu

_TEMPERATURE = 10.0
_BM = 8192


def _flash_body(addr_ref, keys_ref, values_ref, out_ref, m_ref, acc_ref,
                *, num_blocks):
    i = pl.program_id(0)

    @pl.when(i == 0)
    def _init():
        m_ref[...] = jnp.full_like(m_ref, -jnp.inf)
        acc_ref[...] = jnp.zeros_like(acc_ref)

    addr = (addr_ref[...] * _TEMPERATURE).astype(jnp.bfloat16)
    keys = keys_ref[...].astype(jnp.bfloat16)
    vals = values_ref[...].astype(jnp.bfloat16)
    ones = jnp.ones((vals.shape[0], 1), jnp.bfloat16)
    vals1 = jnp.concatenate([vals, ones], axis=1)

    s = jax.lax.dot_general(
        addr, keys,
        (((1,), (1,)), ((), ())),
        preferred_element_type=jnp.float32,
    ).astype(jnp.bfloat16)
    m_prev = m_ref[...]
    m_new = jnp.maximum(m_prev, jnp.max(s, axis=1, keepdims=True))
    alpha = jnp.exp(m_prev - m_new)
    p = jnp.exp(s - m_new)
    pv = jax.lax.dot_general(
        p, vals1,
        (((1,), (0,)), ((), ())),
        preferred_element_type=jnp.float32,
    )
    m_ref[...] = m_new
    acc_ref[...] = acc_ref[...] * alpha.astype(jnp.float32) + pv

    @pl.when(i == num_blocks - 1)
    def _fini():
        acc = acc_ref[...]
        out_ref[...] = acc[:, :-1] / acc[:, -1:]


@jax.jit
def kernel(keys, values, addr):
    M, D = keys.shape
    Q = addr.shape[0]
    V = values.shape[1]
    num_blocks = M // _BM

    return pl.pallas_call(
        functools.partial(_flash_body, num_blocks=num_blocks),
        grid=(num_blocks,),
        in_specs=[
            pl.BlockSpec((Q, D), lambda i: (0, 0)),
            pl.BlockSpec((_BM, D), lambda i: (i, 0)),
            pl.BlockSpec((_BM, V), lambda i: (i, 0)),
        ],
        out_specs=pl.BlockSpec((Q, V), lambda i: (0, 0)),
        out_shape=jax.ShapeDtypeStruct((Q, V), jnp.float32),
        scratch_shapes=[
            pltpu.VMEM((Q, 1), jnp.bfloat16),
            pltpu.VMEM((Q, V + 1), jnp.float32),
        ],
        compiler_params=pltpu.CompilerParams(
            dimension_semantics=("arbitrary",),
        ),
    )(addr, keys, values)
